# baseline (speedup 1.0000x reference)
import functools

import jax
import jax.numpy as jnp
from jax import lax
from jax.experimental import pallas as pl
from jax.experimental.pallas import tpu as pltpu
from jax.experimental.pallas import tpu_sc as plsc

VOCAB = 1000000
HIDDEN = 64
BATCH = 4096
SEQ = 50
NBLK = VOCAB // 8
BLKW = 8 * HIDDEN

NC = 2
NS = 16
NW = NC * NS
L = 16

B_PER_W = BATCH // NW
H = 64
NH = B_PER_W // H


def _make_kernel():
    mesh = plsc.VectorSubcoreMesh(core_axis_name="c", subcore_axis_name="s")

    @functools.partial(
        pl.kernel,
        out_type=jax.ShapeDtypeStruct((SEQ, HIDDEN, BATCH), jnp.float32),
        mesh=mesh,
        scratch_types=[
            pltpu.VMEM((B_PER_W, SEQ), jnp.int32),
            pltpu.VMEM((SEQ, B_PER_W), jnp.int32),
            pltpu.VMEM((SEQ, B_PER_W), jnp.int32),
            pltpu.VMEM((H, BLKW), jnp.float32),
            pltpu.VMEM((H, BLKW), jnp.float32),
            pltpu.VMEM((HIDDEN, B_PER_W), jnp.float32),
            pltpu.SemaphoreType.DMA,
            pltpu.SemaphoreType.DMA,
            pltpu.SemaphoreType.DMA,
        ],
        compiler_params=pltpu.CompilerParams(needs_layout_passes=False),
    )
    def embed(ids_hbm, table_hbm, out_hbm,
              ids_v, blkT_v, offT_v, blk0, blk1, oslab, g0, g1, osem):
        wid = lax.axis_index("s") * NC + lax.axis_index("c")
        base = wid * B_PER_W
        pltpu.sync_copy(ids_hbm.at[pl.ds(base, B_PER_W)], ids_v)

        lanes = lax.iota(jnp.int32, L)

        def tr_ids(s, carry):
            for g in range(B_PER_W // L):
                col = plsc.load_gather(
                    ids_v, [g * L + lanes, jnp.full((L,), s, jnp.int32)])
                blkT_v[s, pl.ds(g * L, L)] = col // 8
                offT_v[s, pl.ds(g * L, L)] = (col % 8) * HIDDEN
            return carry

        lax.fori_loop(0, SEQ, tr_ids, 0)

        blks = (blk0, blk1)
        gsems = (g0, g1)

        def fetch_half(s, h, p):
            pltpu.async_copy(
                table_hbm.at[blkT_v.at[s, pl.ds(h * H, H)]],
                blks[p], gsems[p])

        def drain_half(p):
            pltpu.make_async_copy(
                table_hbm.at[pl.ds(0, H)], blks[p], gsems[p]).wait()

        def transpose_half(s, h, p):
            def tr_g(g, carry):
                off = offT_v[s, pl.ds(h * H + g * L, L)]
                rows = g * L + lanes
                for c in range(HIDDEN):
                    v = plsc.load_gather(
                        blks[p], [rows, off + c])
                    oslab[c, pl.ds(h * H + g * L, L)] = v
                return carry

            lax.fori_loop(0, H // L, tr_g, 0)

        def step(s, carry):
            fetch_half(s, 0, 0)
            fetch_half(s, 1, 1)
            drain_half(0)
            transpose_half(s, 0, 0)
            drain_half(1)
            transpose_half(s, 1, 1)
            pltpu.async_copy(
                oslab, out_hbm.at[s, :, pl.ds(base, B_PER_W)], osem)
            pltpu.make_async_copy(
                oslab, out_hbm.at[0, :, pl.ds(0, B_PER_W)], osem).wait()
            return carry

        lax.fori_loop(0, SEQ, step, 0)

    return embed


_EMBED = _make_kernel()


def kernel(input_ids, word_embeddings):
    table = word_embeddings.reshape(NBLK, BLKW)
    out_t = _EMBED(input_ids.astype(jnp.int32), table)
    return jnp.transpose(out_t, (2, 0, 1))


# speedup vs baseline: 1.5294x; 1.5294x over previous
import functools

import jax
import jax.numpy as jnp
from jax import lax
from jax.experimental import pallas as pl
from jax.experimental.pallas import tpu as pltpu
from jax.experimental.pallas import tpu_sc as plsc

VOCAB = 1000000
HIDDEN = 64
PADW = 128
BATCH = 4096
SEQ = 50

NC = 2
NS = 16
NW = NC * NS

B_PER_W = BATCH // NW
G = 4
NG = B_PER_W // G


def _make_kernel():
    mesh = plsc.VectorSubcoreMesh(core_axis_name="c", subcore_axis_name="s")

    @functools.partial(
        pl.kernel,
        out_type=jax.ShapeDtypeStruct((BATCH, SEQ, HIDDEN), jnp.float32),
        mesh=mesh,
        scratch_types=[
            pltpu.VMEM((B_PER_W, SEQ), jnp.int32),
            pltpu.VMEM((G, SEQ, PADW), jnp.float32),
            pltpu.VMEM((G, SEQ, PADW), jnp.float32),
            pltpu.SemaphoreType.DMA,
            pltpu.SemaphoreType.DMA,
            pltpu.SemaphoreType.DMA,
            pltpu.SemaphoreType.DMA,
        ],
        compiler_params=pltpu.CompilerParams(use_tc_tiling_on_sc=False),
    )
    def embed(ids_hbm, table_hbm, out_hbm, idx_v, rows0, rows1, g0, g1, o0, o1):
        wid = lax.axis_index("s") * NC + lax.axis_index("c")
        base = wid * B_PER_W
        pltpu.sync_copy(ids_hbm.at[pl.ds(base, B_PER_W)], idx_v)

        bufs = (rows0, rows1)
        gsems = (g0, g1)
        osems = (o0, o1)

        def issue_gathers(s, b):
            return [
                pltpu.async_copy(
                    table_hbm.at[idx_v.at[s * G + k]],
                    bufs[b].at[k],
                    gsems[b])
                for k in range(G)
            ]

        pend_g = [issue_gathers(0, 0), issue_gathers(1, 1)]
        pend_o = [None, None]
        for s in range(NG):
            b = s % 2
            for h in pend_g[b]:
                h.wait()
            pend_o[b] = pltpu.async_copy(
                bufs[b].at[:, :, pl.ds(0, HIDDEN)],
                out_hbm.at[pl.ds(base + s * G, G)], osems[b])
            if s + 2 < NG:
                pend_o[b].wait()
                pend_g[b] = issue_gathers(s + 2, b)
        pend_o[NG % 2].wait()
        pend_o[(NG - 1) % 2].wait()

    return embed


_EMBED = _make_kernel()


def kernel(input_ids, word_embeddings):
    table = jnp.pad(word_embeddings, ((0, 0), (0, PADW - HIDDEN)))
    return _EMBED(input_ids.astype(jnp.int32), table)


# speedup vs baseline: 1.5305x; 1.0007x over previous
import functools

import jax
import jax.numpy as jnp
from jax import lax
from jax.experimental import pallas as pl
from jax.experimental.pallas import tpu as pltpu
from jax.experimental.pallas import tpu_sc as plsc

VOCAB = 1000000
HIDDEN = 64
PADW = 128
BATCH = 4096
SEQ = 50

NC = 2
NS = 16
NW = NC * NS

B_PER_W = BATCH // NW
G = 8
NG = B_PER_W // G


def _make_kernel():
    mesh = plsc.VectorSubcoreMesh(core_axis_name="c", subcore_axis_name="s")

    @functools.partial(
        pl.kernel,
        out_type=jax.ShapeDtypeStruct((BATCH, SEQ, HIDDEN), jnp.float32),
        mesh=mesh,
        scratch_types=[
            pltpu.VMEM((B_PER_W, SEQ), jnp.int32),
            pltpu.VMEM((G, SEQ, PADW), jnp.float32),
            pltpu.VMEM((G, SEQ, PADW), jnp.float32),
            pltpu.SemaphoreType.DMA,
            pltpu.SemaphoreType.DMA,
            pltpu.SemaphoreType.DMA,
            pltpu.SemaphoreType.DMA,
        ],
        compiler_params=pltpu.CompilerParams(use_tc_tiling_on_sc=False),
    )
    def embed(ids_hbm, table_hbm, out_hbm, idx_v, rows0, rows1, g0, g1, o0, o1):
        wid = lax.axis_index("s") * NC + lax.axis_index("c")
        base = wid * B_PER_W
        pltpu.sync_copy(ids_hbm.at[pl.ds(base, B_PER_W)], idx_v)

        bufs = (rows0, rows1)
        gsems = (g0, g1)
        osems = (o0, o1)

        def issue_gathers(s, b):
            return [
                pltpu.async_copy(
                    table_hbm.at[idx_v.at[s * G + k]],
                    bufs[b].at[k],
                    gsems[b])
                for k in range(G)
            ]

        pend_g = [issue_gathers(0, 0), issue_gathers(1, 1)]
        pend_o = [None, None]
        for s in range(NG):
            b = s % 2
            for h in pend_g[b]:
                h.wait()
            pend_o[b] = pltpu.async_copy(
                bufs[b].at[:, :, pl.ds(0, HIDDEN)],
                out_hbm.at[pl.ds(base + s * G, G)], osems[b])
            if s + 2 < NG:
                pend_o[b].wait()
                pend_g[b] = issue_gathers(s + 2, b)
        pend_o[NG % 2].wait()
        pend_o[(NG - 1) % 2].wait()

    return embed


_EMBED = _make_kernel()


def kernel(input_ids, word_embeddings):
    table = jnp.pad(word_embeddings, ((0, 0), (0, PADW - HIDDEN)))
    return _EMBED(input_ids.astype(jnp.int32), table)
